# baseline (speedup 1.0000x reference)
import functools

import jax
import jax.numpy as jnp
from jax import lax
from jax.experimental import pallas as pl
from jax.experimental.pallas import tpu as pltpu
from jax.experimental.pallas import tpu_sc as plsc

NNZ = 2097152
NBINS = 2048
NC = 2
NS = 16
NW = NC * NS
CHUNK = NNZ // NW
SUB = 8192
NSUB = CHUNK // SUB
VPS = SUB // 16
NVB = NBINS // 16
MASK21 = (1 << 21) - 1

_MESH = plsc.VectorSubcoreMesh(core_axis_name="c", subcore_axis_name="s")
_CP = pltpu.CompilerParams(needs_layout_passes=False)


def _wid():
    return lax.axis_index("s") * NC + lax.axis_index("c")


def _iota16():
    return lax.iota(jnp.int32, 16)


def _vsl(j):
    return pl.ds(pl.multiple_of(j * 16, 16), 16)


def _zero(ref, nv):
    def body(j, _):
        ref[_vsl(j)] = jnp.zeros((16,), jnp.int32)
        return 0
    lax.fori_loop(0, nv, body, 0)


def _hist_common(src_hbm, out_hbm, fbuf, hist_v, wpe, dig):
    w = _wid()
    _zero(hist_v, NVB)
    ones = jnp.ones((16,), jnp.int32)
    iota = _iota16()

    def sub(s, _):
        start = (w * CHUNK + s * SUB) * wpe
        with jax.named_scope("h_streamin"):
            pltpu.sync_copy(
                src_hbm.at[pl.ds(pl.multiple_of(start, 8), SUB * wpe)], fbuf)

        def body(j, _):
            d = dig(fbuf, j, iota)
            plsc.addupdate_scatter(hist_v, [d], ones)
            return 0
        with jax.named_scope("h_compute"):
            lax.fori_loop(0, VPS, body, 0)
        return 0
    lax.fori_loop(0, NSUB, sub, 0)
    pltpu.sync_copy(hist_v, out_hbm.at[w])


def _dig_a(fbuf, j, iota):
    lidx = j * 16 + iota
    return plsc.load_gather(fbuf, [lidx * 3])


def _dig_b(fbuf, j, iota):
    return fbuf[_vsl(j)] & (NBINS - 1)


def _dig_c(fbuf, j, iota):
    return (fbuf[_vsl(j)] >> 21) & (NBINS - 1)


def _offsets(hist_hbm, row_v, tot_v, off_v, spl_ref=None):
    w = _wid()
    _zero(tot_v, NVB)
    _zero(off_v, NVB)

    def trow(t, _):
        pltpu.sync_copy(hist_hbm.at[t], row_v)
        m = (t < w).astype(jnp.int32)

        def inner(j, _):
            sl = _vsl(j)
            row = row_v[sl]
            tot_v[sl] = tot_v[sl] + row
            off_v[sl] = off_v[sl] + row * m
            return 0
        lax.fori_loop(0, NVB, inner, 0)
        return 0
    lax.fori_loop(0, NW, trow, 0)

    fifteen = jnp.full((16,), 15, jnp.int32)

    def scan(j, carry):
        sl = _vsl(j)
        v = tot_v[sl]
        cs = plsc.cumsum(v)
        excl = cs - v + carry
        if spl_ref is not None:
            spl_ref[sl] = excl
        off_v[sl] = off_v[sl] + excl
        return carry + jnp.take(cs, fifteen)
    lax.fori_loop(0, NVB, scan, jnp.zeros((16,), jnp.int32))


def _rank(off_v, d):
    cnt, lastm = plsc.scan_count(d)
    cnt = cnt.astype(jnp.int32)
    cur = plsc.load_gather(off_v, [d])
    plsc.store_scatter(off_v, [d], cur + cnt, mask=lastm)
    return cur + cnt - 1


@functools.partial(
    pl.kernel, mesh=_MESH, compiler_params=_CP,
    out_type=jax.ShapeDtypeStruct((NW, NBINS), jnp.int32),
    scratch_types=[pltpu.VMEM((3 * SUB,), jnp.int32),
                   pltpu.VMEM((NBINS,), jnp.int32)],
)
def _hist_a(F, out, fbuf, hist_v):
    _hist_common(F, out, fbuf, hist_v, 3, _dig_a)


@functools.partial(
    pl.kernel, mesh=_MESH, compiler_params=_CP,
    out_type=jax.ShapeDtypeStruct((NW, NBINS), jnp.int32),
    scratch_types=[pltpu.VMEM((SUB,), jnp.int32),
                   pltpu.VMEM((NBINS,), jnp.int32)],
)
def _hist_b(K1, out, fbuf, hist_v):
    _hist_common(K1, out, fbuf, hist_v, 1, _dig_b)


@functools.partial(
    pl.kernel, mesh=_MESH, compiler_params=_CP,
    out_type=jax.ShapeDtypeStruct((NW, NBINS), jnp.int32),
    scratch_types=[pltpu.VMEM((SUB,), jnp.int32),
                   pltpu.VMEM((NBINS,), jnp.int32)],
)
def _hist_c(P2, out, fbuf, hist_v):
    _hist_common(P2, out, fbuf, hist_v, 1, _dig_c)


@functools.partial(
    pl.kernel, mesh=_MESH, compiler_params=_CP,
    out_type=(jax.ShapeDtypeStruct((NNZ,), jnp.int32),
              jax.ShapeDtypeStruct((NNZ,), jnp.int32)),
    scratch_types=[pltpu.VMEM((3 * SUB,), jnp.int32),
                   pltpu.VMEM((SUB,), jnp.int32),
                   pltpu.VMEM((SUB,), jnp.int32),
                   pltpu.VMEM((SUB,), jnp.int32),
                   pltpu.VMEM((NBINS,), jnp.int32),
                   pltpu.VMEM((NBINS,), jnp.int32),
                   pltpu.VMEM((NBINS,), jnp.int32),
                   pltpu.VMEM((NBINS,), jnp.int32),
                   pltpu.SemaphoreType.DMA],
)
def _perm_a(F, hA, K1, I1, fbuf, kbuf, ibuf, dbuf, row_v, tot_v, off_v,
            spl_v, sem):
    w = _wid()
    _offsets(hA, row_v, tot_v, off_v, spl_v)
    iota = _iota16()

    def sub(s, _):
        base = w * CHUNK + s * SUB
        pltpu.sync_copy(
            F.at[pl.ds(pl.multiple_of(base * 3, 8), SUB * 3)], fbuf)

        def body(j, _):
            lidx = j * 16 + iota
            b = plsc.load_gather(fbuf, [lidx * 3])
            r = plsc.load_gather(fbuf, [lidx * 3 + 1])
            k2 = r + plsc.load_gather(spl_v, [b])
            sl = _vsl(j)
            dbuf[sl] = _rank(off_v, b)
            kbuf[sl] = k2
            ibuf[sl] = base + lidx
            return 0
        lax.fori_loop(0, VPS, body, 0)
        pltpu.async_copy(kbuf, K1.at[dbuf], sem).wait()
        pltpu.async_copy(ibuf, I1.at[dbuf], sem).wait()
        return 0
    lax.fori_loop(0, NSUB, sub, 0)


@functools.partial(
    pl.kernel, mesh=_MESH, compiler_params=_CP,
    out_type=jax.ShapeDtypeStruct((NNZ,), jnp.int32),
    scratch_types=[pltpu.VMEM((SUB,), jnp.int32),
                   pltpu.VMEM((SUB,), jnp.int32),
                   pltpu.VMEM((SUB,), jnp.int32),
                   pltpu.VMEM((SUB,), jnp.int32),
                   pltpu.VMEM((NBINS,), jnp.int32),
                   pltpu.VMEM((NBINS,), jnp.int32),
                   pltpu.VMEM((NBINS,), jnp.int32),
                   pltpu.SemaphoreType.DMA],
)
def _perm_b(K1, I1, hB, P2, kfbuf, ifbuf, pbuf, dbuf, row_v, tot_v, off_v,
            sem):
    w = _wid()
    with jax.named_scope("pb_offsets"):
        _offsets(hB, row_v, tot_v, off_v)

    def sub(s, _):
        base = w * CHUNK + s * SUB
        with jax.named_scope("pb_streamin"):
            pltpu.sync_copy(K1.at[pl.ds(pl.multiple_of(base, 8), SUB)], kfbuf)
            pltpu.sync_copy(I1.at[pl.ds(pl.multiple_of(base, 8), SUB)], ifbuf)

        def body(j, _):
            sl = _vsl(j)
            x = kfbuf[sl]
            d = x & (NBINS - 1)
            dbuf[sl] = _rank(off_v, d)
            pbuf[sl] = ((x >> 11) << 21) | ifbuf[sl]
            return 0
        with jax.named_scope("pb_compute"):
            lax.fori_loop(0, VPS, body, 0)
        with jax.named_scope("pb_scatter"):
            pltpu.async_copy(pbuf, P2.at[dbuf], sem).wait()
        return 0
    lax.fori_loop(0, NSUB, sub, 0)


@functools.partial(
    pl.kernel, mesh=_MESH, compiler_params=_CP,
    out_type=jax.ShapeDtypeStruct((NNZ,), jnp.int32),
    scratch_types=[pltpu.VMEM((SUB,), jnp.int32),
                   pltpu.VMEM((SUB,), jnp.int32),
                   pltpu.VMEM((SUB,), jnp.int32),
                   pltpu.VMEM((NBINS,), jnp.int32),
                   pltpu.VMEM((NBINS,), jnp.int32),
                   pltpu.VMEM((NBINS,), jnp.int32),
                   pltpu.SemaphoreType.DMA],
)
def _perm_c(P2, hC, I3, pfbuf, obuf, dbuf, row_v, tot_v, off_v, sem):
    w = _wid()
    _offsets(hC, row_v, tot_v, off_v)

    def sub(s, _):
        base = w * CHUNK + s * SUB
        pltpu.sync_copy(P2.at[pl.ds(pl.multiple_of(base, 8), SUB)], pfbuf)

        def body(j, _):
            sl = _vsl(j)
            x = pfbuf[sl]
            d = (x >> 21) & (NBINS - 1)
            dbuf[sl] = _rank(off_v, d)
            obuf[sl] = x & MASK21
            return 0
        lax.fori_loop(0, VPS, body, 0)
        pltpu.async_copy(obuf, I3.at[dbuf], sem).wait()
        return 0
    lax.fori_loop(0, NSUB, sub, 0)


@functools.partial(
    pl.kernel, mesh=_MESH, compiler_params=_CP,
    out_type=(jax.ShapeDtypeStruct((2 * NNZ,), jnp.int32),
              jax.ShapeDtypeStruct((NNZ,), jnp.int32),
              jax.ShapeDtypeStruct((NNZ,), jnp.float32)),
    scratch_types=[pltpu.VMEM((SUB,), jnp.int32),
                   pltpu.VMEM((SUB,), jnp.int32),
                   pltpu.VMEM((SUB,), jnp.int32),
                   pltpu.VMEM((SUB,), jnp.int32),
                   pltpu.VMEM((SUB,), jnp.int32),
                   pltpu.VMEM((SUB,), jnp.int32),
                   pltpu.VMEM((SUB,), jnp.int32),
                   pltpu.VMEM((SUB,), jnp.float32),
                   pltpu.VMEM((2 * SUB,), jnp.int32),
                   pltpu.SemaphoreType.DMA],
)
def _final(I3, F, V, EI2, RID, EW, ibuf, g0, g1, g2, bb, rb, cb, vb, eibuf,
           sem):
    w = _wid()
    iota = _iota16()

    def sub(s, _):
        base = w * CHUNK + s * SUB
        pltpu.sync_copy(I3.at[pl.ds(pl.multiple_of(base, 8), SUB)], ibuf)

        def mkidx(j, _):
            sl = _vsl(j)
            t3 = ibuf[sl] * 3
            g0[sl] = t3
            g1[sl] = t3 + 1
            g2[sl] = t3 + 2
            return 0
        lax.fori_loop(0, VPS, mkidx, 0)
        with jax.named_scope("f_gather"):
            c0 = pltpu.async_copy(F.at[g0], bb, sem)
            c1 = pltpu.async_copy(F.at[g1], rb, sem)
            c2 = pltpu.async_copy(F.at[g2], cb, sem)
            c3 = pltpu.async_copy(V.at[ibuf], vb, sem)
            c0.wait()
            c1.wait()
            c2.wait()
            c3.wait()

        def ilv(j, _):
            sl = _vsl(j)
            lidx = j * 16 + iota
            plsc.store_scatter(eibuf, [lidx * 2], rb[sl])
            plsc.store_scatter(eibuf, [lidx * 2 + 1], cb[sl])
            return 0
        lax.fori_loop(0, VPS, ilv, 0)
        pltpu.sync_copy(
            eibuf, EI2.at[pl.ds(pl.multiple_of(base * 2, 8), SUB * 2)])
        pltpu.sync_copy(bb, RID.at[pl.ds(pl.multiple_of(base, 8), SUB)])
        pltpu.sync_copy(vb, EW.at[pl.ds(pl.multiple_of(base, 8), SUB)])
        return 0
    lax.fori_loop(0, NSUB, sub, 0)


def kernel(indices, values):
    F = indices.reshape(3 * NNZ)
    hA = _hist_a(F)
    K1, I1 = _perm_a(F, hA)
    hB = _hist_b(K1)
    P2 = _perm_b(K1, I1, hB)
    hC = _hist_c(P2)
    I3 = _perm_c(P2, hC)
    EI2, RID, EW = _final(I3, F, values)
    return EI2.reshape(NNZ, 2), RID, EW[:, None]


# speedup vs baseline: 1.5649x; 1.5649x over previous
import functools

import jax
import jax.numpy as jnp
from jax import lax
from jax.experimental import pallas as pl
from jax.experimental.pallas import tpu as pltpu
from jax.experimental.pallas import tpu_sc as plsc

NNZ = 2097152
NBINS = 2048
NC = 2
NS = 16
NW = NC * NS
CHUNK = NNZ // NW
SUB = 8192
NSUB = CHUNK // SUB
VPS = SUB // 16
NVB = NBINS // 16
MASK21 = (1 << 21) - 1

_MESH = plsc.VectorSubcoreMesh(core_axis_name="c", subcore_axis_name="s")
_CP = pltpu.CompilerParams(needs_layout_passes=False)


def _wid():
    return lax.axis_index("s") * NC + lax.axis_index("c")


def _vsl(j):
    return pl.ds(pl.multiple_of(j * 16, 16), 16)


def _zero(ref, nv):
    def body(j, _):
        ref[_vsl(j)] = jnp.zeros((16,), jnp.int32)
        return 0
    lax.fori_loop(0, nv, body, 0)


def _chunk(arr, base, n):
    return arr.at[pl.ds(pl.multiple_of(base, 8), n)]


def _hist_common(src_hbm, out_hbm, fbuf, hist_v, dig):
    w = _wid()
    _zero(hist_v, NVB)
    ones = jnp.ones((16,), jnp.int32)

    def sub(s, _):
        pltpu.sync_copy(_chunk(src_hbm, w * CHUNK + s * SUB, SUB), fbuf)

        def body(j, _):
            d = dig(fbuf[_vsl(j)])
            plsc.addupdate_scatter(hist_v, [d], ones)
            return 0
        lax.fori_loop(0, VPS, body, 0)
        return 0
    lax.fori_loop(0, NSUB, sub, 0)
    pltpu.sync_copy(hist_v, out_hbm.at[w])


def _dig_id(x):
    return x & (NBINS - 1)


def _dig_hi(x):
    return (x >> 21) & (NBINS - 1)


def _offsets(hist_hbm, row_v, tot_v, off_v, spl_ref=None):
    w = _wid()
    _zero(tot_v, NVB)
    _zero(off_v, NVB)

    def trow(t, _):
        pltpu.sync_copy(hist_hbm.at[t], row_v)
        m = (t < w).astype(jnp.int32)

        def inner(j, _):
            sl = _vsl(j)
            row = row_v[sl]
            tot_v[sl] = tot_v[sl] + row
            off_v[sl] = off_v[sl] + row * m
            return 0
        lax.fori_loop(0, NVB, inner, 0)
        return 0
    lax.fori_loop(0, NW, trow, 0)

    fifteen = jnp.full((16,), 15, jnp.int32)

    def scan(j, carry):
        sl = _vsl(j)
        v = tot_v[sl]
        cs = plsc.cumsum(v)
        excl = cs - v + carry
        if spl_ref is not None:
            spl_ref[sl] = excl
        off_v[sl] = off_v[sl] + excl
        return carry + jnp.take(cs, fifteen)
    lax.fori_loop(0, NVB, scan, jnp.zeros((16,), jnp.int32))


def _rank(off_v, d):
    cnt, lastm = plsc.scan_count(d)
    cnt = cnt.astype(jnp.int32)
    cur = plsc.load_gather(off_v, [d])
    plsc.store_scatter(off_v, [d], cur + cnt, mask=lastm)
    return cur + cnt - 1


@functools.partial(
    pl.kernel, mesh=_MESH, compiler_params=_CP,
    out_type=jax.ShapeDtypeStruct((NW, NBINS), jnp.int32),
    scratch_types=[pltpu.VMEM((SUB,), jnp.int32),
                   pltpu.VMEM((NBINS,), jnp.int32)],
)
def _hist_a(B, out, fbuf, hist_v):
    _hist_common(B, out, fbuf, hist_v, _dig_id)


@functools.partial(
    pl.kernel, mesh=_MESH, compiler_params=_CP,
    out_type=jax.ShapeDtypeStruct((NW, NBINS), jnp.int32),
    scratch_types=[pltpu.VMEM((SUB,), jnp.int32),
                   pltpu.VMEM((NBINS,), jnp.int32)],
)
def _hist_hi(X, out, fbuf, hist_v):
    _hist_common(X, out, fbuf, hist_v, _dig_hi)


@functools.partial(
    pl.kernel, mesh=_MESH, compiler_params=_CP,
    out_type=(jax.ShapeDtypeStruct((NNZ,), jnp.int32),
              jax.ShapeDtypeStruct((NBINS,), jnp.int32)),
    scratch_types=[pltpu.VMEM((SUB,), jnp.int32),
                   pltpu.VMEM((SUB,), jnp.int32),
                   pltpu.VMEM((SUB,), jnp.int32),
                   pltpu.VMEM((SUB,), jnp.int32),
                   pltpu.VMEM((NBINS,), jnp.int32),
                   pltpu.VMEM((NBINS,), jnp.int32),
                   pltpu.VMEM((NBINS,), jnp.int32),
                   pltpu.VMEM((NBINS,), jnp.int32),
                   pltpu.SemaphoreType.DMA],
)
def _perm_a(B, R, hA, W1, SPL, bbuf, rbuf, wbuf, dbuf, row_v, tot_v, off_v,
            spl_v, sem):
    w = _wid()
    _offsets(hA, row_v, tot_v, off_v, spl_v)
    iota = lax.iota(jnp.int32, 16)

    @pl.when(w == 0)
    def _():
        pltpu.sync_copy(spl_v, SPL)

    def sub(s, _):
        base = w * CHUNK + s * SUB
        pltpu.sync_copy(_chunk(B, base, SUB), bbuf)
        pltpu.sync_copy(_chunk(R, base, SUB), rbuf)

        def body(j, _):
            sl = _vsl(j)
            b = bbuf[sl]
            k2 = rbuf[sl] + plsc.load_gather(spl_v, [b])
            dbuf[sl] = _rank(off_v, b)
            wbuf[sl] = ((k2 & (NBINS - 1)) << 21) | (base + j * 16 + iota)
            return 0
        lax.fori_loop(0, VPS, body, 0)
        pltpu.async_copy(wbuf, W1.at[dbuf], sem).wait()
        return 0
    lax.fori_loop(0, NSUB, sub, 0)


@functools.partial(
    pl.kernel, mesh=_MESH, compiler_params=_CP,
    out_type=jax.ShapeDtypeStruct((NNZ,), jnp.int32),
    scratch_types=[pltpu.VMEM((SUB,), jnp.int32),
                   pltpu.VMEM((SUB,), jnp.int32),
                   pltpu.VMEM((SUB,), jnp.int32),
                   pltpu.VMEM((SUB,), jnp.int32),
                   pltpu.VMEM((SUB,), jnp.int32),
                   pltpu.VMEM((SUB,), jnp.int32),
                   pltpu.VMEM((NBINS,), jnp.int32),
                   pltpu.VMEM((NBINS,), jnp.int32),
                   pltpu.VMEM((NBINS,), jnp.int32),
                   pltpu.VMEM((NBINS,), jnp.int32),
                   pltpu.SemaphoreType.DMA],
)
def _perm_b(W1, B, R, SPL, hB, P2, wbuf, ibuf, bgbuf, rgbuf, pbuf, dbuf,
            row_v, tot_v, off_v, spl_v, sem):
    w = _wid()
    _offsets(hB, row_v, tot_v, off_v)
    pltpu.sync_copy(SPL, spl_v)

    def sub(s, _):
        base = w * CHUNK + s * SUB
        pltpu.sync_copy(_chunk(W1, base, SUB), wbuf)

        def ext(j, _):
            sl = _vsl(j)
            ibuf[sl] = wbuf[sl] & MASK21
            return 0
        lax.fori_loop(0, VPS, ext, 0)
        c0 = pltpu.async_copy(B.at[ibuf], bgbuf, sem)
        c1 = pltpu.async_copy(R.at[ibuf], rgbuf, sem)
        c0.wait()
        c1.wait()

        def body(j, _):
            sl = _vsl(j)
            d = (wbuf[sl] >> 21) & (NBINS - 1)
            k2 = rgbuf[sl] + plsc.load_gather(spl_v, [bgbuf[sl]])
            dbuf[sl] = _rank(off_v, d)
            pbuf[sl] = ((k2 >> 11) << 21) | ibuf[sl]
            return 0
        lax.fori_loop(0, VPS, body, 0)
        pltpu.async_copy(pbuf, P2.at[dbuf], sem).wait()
        return 0
    lax.fori_loop(0, NSUB, sub, 0)


@functools.partial(
    pl.kernel, mesh=_MESH, compiler_params=_CP,
    out_type=jax.ShapeDtypeStruct((NNZ,), jnp.int32),
    scratch_types=[pltpu.VMEM((SUB,), jnp.int32),
                   pltpu.VMEM((SUB,), jnp.int32),
                   pltpu.VMEM((SUB,), jnp.int32),
                   pltpu.VMEM((NBINS,), jnp.int32),
                   pltpu.VMEM((NBINS,), jnp.int32),
                   pltpu.VMEM((NBINS,), jnp.int32),
                   pltpu.SemaphoreType.DMA],
)
def _perm_c(P2, hC, I3, pfbuf, obuf, dbuf, row_v, tot_v, off_v, sem):
    w = _wid()
    _offsets(hC, row_v, tot_v, off_v)

    def sub(s, _):
        base = w * CHUNK + s * SUB
        pltpu.sync_copy(_chunk(P2, base, SUB), pfbuf)

        def body(j, _):
            sl = _vsl(j)
            x = pfbuf[sl]
            d = (x >> 21) & (NBINS - 1)
            dbuf[sl] = _rank(off_v, d)
            obuf[sl] = x & MASK21
            return 0
        lax.fori_loop(0, VPS, body, 0)
        pltpu.async_copy(obuf, I3.at[dbuf], sem).wait()
        return 0
    lax.fori_loop(0, NSUB, sub, 0)


@functools.partial(
    pl.kernel, mesh=_MESH, compiler_params=_CP,
    out_type=(jax.ShapeDtypeStruct((2 * NNZ,), jnp.int32),
              jax.ShapeDtypeStruct((NNZ,), jnp.int32),
              jax.ShapeDtypeStruct((NNZ,), jnp.float32)),
    scratch_types=[pltpu.VMEM((SUB,), jnp.int32),
                   pltpu.VMEM((SUB,), jnp.int32),
                   pltpu.VMEM((SUB,), jnp.int32),
                   pltpu.VMEM((SUB,), jnp.int32),
                   pltpu.VMEM((SUB,), jnp.float32),
                   pltpu.VMEM((2 * SUB,), jnp.int32),
                   pltpu.SemaphoreType.DMA],
)
def _final(I3, B, R, C, V, EI2, RID, EW, ibuf, bb, rb, cb, vb, eibuf, sem):
    w = _wid()
    iota = lax.iota(jnp.int32, 16)

    def sub(s, _):
        base = w * CHUNK + s * SUB
        pltpu.sync_copy(_chunk(I3, base, SUB), ibuf)
        c0 = pltpu.async_copy(B.at[ibuf], bb, sem)
        c1 = pltpu.async_copy(R.at[ibuf], rb, sem)
        c2 = pltpu.async_copy(C.at[ibuf], cb, sem)
        c3 = pltpu.async_copy(V.at[ibuf], vb, sem)
        c0.wait()
        c1.wait()
        c2.wait()
        c3.wait()

        def ilv(j, _):
            sl = _vsl(j)
            lidx = j * 16 + iota
            plsc.store_scatter(eibuf, [lidx * 2], rb[sl])
            plsc.store_scatter(eibuf, [lidx * 2 + 1], cb[sl])
            return 0
        lax.fori_loop(0, VPS, ilv, 0)
        pltpu.sync_copy(eibuf, _chunk(EI2, base * 2, SUB * 2))
        pltpu.sync_copy(bb, _chunk(RID, base, SUB))
        pltpu.sync_copy(vb, _chunk(EW, base, SUB))
        return 0
    lax.fori_loop(0, NSUB, sub, 0)


def kernel(indices, values):
    B = indices[:, 0]
    R = indices[:, 1]
    C = indices[:, 2]
    hA = _hist_a(B)
    W1, SPL = _perm_a(B, R, hA)
    hB = _hist_hi(W1)
    P2 = _perm_b(W1, B, R, SPL, hB)
    hC = _hist_hi(P2)
    I3 = _perm_c(P2, hC)
    EI2, RID, EW = _final(I3, B, R, C, values)
    return EI2.reshape(NNZ, 2), RID, EW[:, None]
